# baseline (speedup 1.0000x reference)
import jax
import jax.numpy as jnp
import numpy as np
from jax import lax
from jax.experimental import pallas as pl
from jax.experimental.pallas import tpu as pltpu
from jax.experimental.pallas import tpu_sc as plsc

N_NODES = 10000
D = 128
E_EDGES = 320000

NUM_CORES = 2
NUM_SUBCORES = 16
NUM_WORKERS = NUM_CORES * NUM_SUBCORES
EDGES_PER_W = E_EDGES // NUM_WORKERS
CHUNK = 40
CHUNKS_PER_W = EDGES_PER_W // CHUNK
ACC_ROWS = 10240
ROWS_PER_TILE = ACC_ROWS // NUM_SUBCORES

NBUF = 8
ROUNDS = CHUNKS_PER_W // NBUF
TAIL = CHUNKS_PER_W - ROUNDS * NBUF


def _sc_body(x_hbm, src_hbm, dst_hbm, zero_hbm, part_hbm, *scr):
    sidx = scr[0:NBUF]
    didx = scr[NBUF:2 * NBUF]
    rows = scr[2 * NBUF:3 * NBUF]
    acc_sh = scr[3 * NBUF]
    isems = scr[3 * NBUF + 1:4 * NBUF + 1]
    gsems = scr[4 * NBUF + 1:5 * NBUF + 1]
    c = lax.axis_index("c")
    s = lax.axis_index("s")
    g = c * NUM_SUBCORES + s
    e0 = g * EDGES_PER_W

    def ifetch(i, j):
        pltpu.async_copy(src_hbm.at[pl.ds(e0 + i * CHUNK, CHUNK)], sidx[j], isems[j])
        pltpu.async_copy(dst_hbm.at[pl.ds(e0 + i * CHUNK, CHUNK)], didx[j], isems[j])

    def drain_idx(j):
        pltpu.make_async_copy(src_hbm.at[pl.ds(0, CHUNK)], sidx[j], isems[j]).wait()
        pltpu.make_async_copy(dst_hbm.at[pl.ds(0, CHUNK)], didx[j], isems[j]).wait()

    def gather(j):
        pltpu.async_copy(x_hbm.at[sidx[j]], rows[j], gsems[j])

    def drain_gather(j):
        pltpu.make_async_copy(x_hbm.at[pl.ds(0, CHUNK)], rows[j], gsems[j]).wait()

    def scatter(j):
        pltpu.sync_copy(rows[j], acc_sh.at[didx[j]], add=True)

    for j in range(NBUF):
        ifetch(j, j)
    row0 = s * ROWS_PER_TILE
    pltpu.sync_copy(zero_hbm.at[pl.ds(row0, ROWS_PER_TILE)],
                    acc_sh.at[pl.ds(row0, ROWS_PER_TILE)])
    plsc.subcore_barrier()
    for j in range(NBUF):
        drain_idx(j)
        gather(j)

    def round_fn(k, carry):
        i0 = k * NBUF
        for j in range(NBUF):
            drain_gather(j)
            scatter(j)
            nxt = i0 + NBUF + j

            @pl.when(nxt < CHUNKS_PER_W)
            def _():
                ifetch(nxt, j)
            if j > 0:
                @pl.when(i0 + NBUF + j - 1 < CHUNKS_PER_W)
                def _():
                    drain_idx(j - 1)
                    gather(j - 1)

        @pl.when(i0 + 2 * NBUF - 1 < CHUNKS_PER_W)
        def _():
            drain_idx(NBUF - 1)
            gather(NBUF - 1)
        return carry

    lax.fori_loop(0, ROUNDS, round_fn, 0)
    for j in range(TAIL):
        drain_gather(j)
        scatter(j)
    plsc.subcore_barrier()

    pltpu.sync_copy(acc_sh.at[pl.ds(row0, ROWS_PER_TILE)],
                    part_hbm.at[c, pl.ds(row0, ROWS_PER_TILE)])


@jax.jit
def _sc_aggregate(x, src_r, dst_r):
    zeros = jnp.asarray(np.zeros((ACC_ROWS, D), dtype=np.float32))
    mesh = plsc.VectorSubcoreMesh(core_axis_name="c", subcore_axis_name="s")
    return pl.kernel(
        _sc_body,
        out_type=jax.ShapeDtypeStruct((NUM_CORES, ACC_ROWS, D), jnp.float32),
        mesh=mesh,
        scratch_types=(
            [pltpu.VMEM((CHUNK,), jnp.int32)] * (2 * NBUF)
            + [pltpu.VMEM((CHUNK, D), jnp.float32)] * NBUF
            + [pltpu.VMEM_SHARED((ACC_ROWS, D), jnp.float32)]
            + [pltpu.SemaphoreType.DMA] * (2 * NBUF)
        ),
    )(x, src_r, dst_r, zeros)


ROW_BLK = 2000


def _tc_body(p_ref, x_ref, wrel_ref, wroot_ref, b_ref, o_ref):
    agg = p_ref[0] + p_ref[1]
    dn = (((1,), (1,)), ((), ()))
    o_ref[...] = (
        lax.dot_general(agg, wrel_ref[...], dn, preferred_element_type=jnp.float32)
        + lax.dot_general(x_ref[...], wroot_ref[...], dn, preferred_element_type=jnp.float32)
        + b_ref[...]
    )


@jax.jit
def _tc_combine(parts, x, wrel_t, wroot_t, b2):
    grid = N_NODES // ROW_BLK
    return pl.pallas_call(
        _tc_body,
        grid=(grid,),
        in_specs=[
            pl.BlockSpec((NUM_CORES, ROW_BLK, D), lambda i: (0, i, 0)),
            pl.BlockSpec((ROW_BLK, D), lambda i: (i, 0)),
            pl.BlockSpec((D, D), lambda i: (0, 0)),
            pl.BlockSpec((D, D), lambda i: (0, 0)),
            pl.BlockSpec((1, D), lambda i: (0, 0)),
        ],
        out_specs=pl.BlockSpec((ROW_BLK, D), lambda i: (i, 0)),
        out_shape=jax.ShapeDtypeStruct((N_NODES, D), jnp.float32),
    )(parts, x, wrel_t, wroot_t, b2)


def kernel(x, edge_index, W_rel, W_root, b):
    parts = _sc_aggregate(x, edge_index[0], edge_index[1])
    return _tc_combine(parts, x, W_rel, W_root, b.reshape(1, D))


# speedup vs baseline: 1.2291x; 1.2291x over previous
import jax
import jax.numpy as jnp
import numpy as np
from jax import lax
from jax.experimental import pallas as pl
from jax.experimental.pallas import tpu as pltpu
from jax.experimental.pallas import tpu_sc as plsc

N_NODES = 10000
D = 128
E_EDGES = 320000

NUM_CORES = 2
NUM_SUBCORES = 16
NUM_WORKERS = NUM_CORES * NUM_SUBCORES
EDGES_PER_W = E_EDGES // NUM_WORKERS
CHUNK = 80
CHUNKS_PER_W = EDGES_PER_W // CHUNK
ACC_ROWS = 10240
ROWS_PER_TILE = ACC_ROWS // NUM_SUBCORES

NBUF = 4
ROUNDS = CHUNKS_PER_W // NBUF
TAIL = CHUNKS_PER_W - ROUNDS * NBUF


def _sc_body(x_hbm, src_hbm, dst_hbm, zero_hbm, part_hbm, *scr):
    sidx = scr[0:NBUF]
    didx = scr[NBUF:2 * NBUF]
    rows = scr[2 * NBUF:3 * NBUF]
    acc_sh = scr[3 * NBUF]
    isems = scr[3 * NBUF + 1:4 * NBUF + 1]
    gsems = scr[4 * NBUF + 1:5 * NBUF + 1]
    c = lax.axis_index("c")
    s = lax.axis_index("s")
    g = c * NUM_SUBCORES + s
    e0 = g * EDGES_PER_W

    def ifetch(i, j):
        pltpu.async_copy(src_hbm.at[pl.ds(e0 + i * CHUNK, CHUNK)], sidx[j], isems[j])
        pltpu.async_copy(dst_hbm.at[pl.ds(e0 + i * CHUNK, CHUNK)], didx[j], isems[j])

    def drain_idx(j):
        pltpu.make_async_copy(src_hbm.at[pl.ds(0, CHUNK)], sidx[j], isems[j]).wait()
        pltpu.make_async_copy(dst_hbm.at[pl.ds(0, CHUNK)], didx[j], isems[j]).wait()

    def gather(j):
        pltpu.async_copy(x_hbm.at[sidx[j]], rows[j], gsems[j])

    def drain_gather(j):
        pltpu.make_async_copy(x_hbm.at[pl.ds(0, CHUNK)], rows[j], gsems[j]).wait()

    def scatter(j):
        pltpu.sync_copy(rows[j], acc_sh.at[didx[j]], add=True)

    for j in range(NBUF):
        ifetch(j, j)
    row0 = s * ROWS_PER_TILE
    pltpu.sync_copy(zero_hbm.at[pl.ds(row0, ROWS_PER_TILE)],
                    acc_sh.at[pl.ds(row0, ROWS_PER_TILE)])
    plsc.subcore_barrier()
    for j in range(NBUF):
        drain_idx(j)
        gather(j)

    def round_fn(k, carry):
        i0 = k * NBUF
        for j in range(NBUF):
            drain_gather(j)
            scatter(j)
            nxt = i0 + NBUF + j

            @pl.when(nxt < CHUNKS_PER_W)
            def _():
                ifetch(nxt, j)
            if j > 0:
                @pl.when(i0 + NBUF + j - 1 < CHUNKS_PER_W)
                def _():
                    drain_idx(j - 1)
                    gather(j - 1)

        @pl.when(i0 + 2 * NBUF - 1 < CHUNKS_PER_W)
        def _():
            drain_idx(NBUF - 1)
            gather(NBUF - 1)
        return carry

    lax.fori_loop(0, ROUNDS, round_fn, 0)
    for j in range(TAIL):
        drain_gather(j)
        scatter(j)
    plsc.subcore_barrier()

    pltpu.sync_copy(acc_sh.at[pl.ds(row0, ROWS_PER_TILE)],
                    part_hbm.at[c, pl.ds(row0, ROWS_PER_TILE)])


@jax.jit
def _sc_aggregate(x, src_r, dst_r):
    zeros = jnp.asarray(np.zeros((ACC_ROWS, D), dtype=np.float32))
    mesh = plsc.VectorSubcoreMesh(core_axis_name="c", subcore_axis_name="s")
    return pl.kernel(
        _sc_body,
        out_type=jax.ShapeDtypeStruct((NUM_CORES, ACC_ROWS, D), jnp.float32),
        mesh=mesh,
        scratch_types=(
            [pltpu.VMEM((CHUNK,), jnp.int32)] * (2 * NBUF)
            + [pltpu.VMEM((CHUNK, D), jnp.float32)] * NBUF
            + [pltpu.VMEM_SHARED((ACC_ROWS, D), jnp.float32)]
            + [pltpu.SemaphoreType.DMA] * (2 * NBUF)
        ),
    )(x, src_r, dst_r, zeros)


SPLIT_BLK = 32000


def _split_body(ei_ref, src_ref, dst_ref):
    src_ref[...] = ei_ref[0]
    dst_ref[...] = ei_ref[1]


@jax.jit
def _edge_split(edge_index):
    return pl.pallas_call(
        _split_body,
        grid=(1,),
        in_specs=[pl.BlockSpec((2, E_EDGES), lambda i: (0, 0))],
        out_specs=[
            pl.BlockSpec((E_EDGES,), lambda i: (0,)),
            pl.BlockSpec((E_EDGES,), lambda i: (0,)),
        ],
        out_shape=[
            jax.ShapeDtypeStruct((E_EDGES,), jnp.int32),
            jax.ShapeDtypeStruct((E_EDGES,), jnp.int32),
        ],
    )(edge_index)


ROW_BLK = 2000


def _tc_body(p_ref, x_ref, wrel_ref, wroot_ref, b_ref, o_ref):
    agg = p_ref[0] + p_ref[1]
    dn = (((1,), (1,)), ((), ()))
    o_ref[...] = (
        lax.dot_general(agg, wrel_ref[...], dn, preferred_element_type=jnp.float32)
        + lax.dot_general(x_ref[...], wroot_ref[...], dn, preferred_element_type=jnp.float32)
        + b_ref[...]
    )


@jax.jit
def _tc_combine(parts, x, wrel_t, wroot_t, b2):
    grid = N_NODES // ROW_BLK
    return pl.pallas_call(
        _tc_body,
        grid=(grid,),
        in_specs=[
            pl.BlockSpec((NUM_CORES, ROW_BLK, D), lambda i: (0, i, 0)),
            pl.BlockSpec((ROW_BLK, D), lambda i: (i, 0)),
            pl.BlockSpec((D, D), lambda i: (0, 0)),
            pl.BlockSpec((D, D), lambda i: (0, 0)),
            pl.BlockSpec((1, D), lambda i: (0, 0)),
        ],
        out_specs=pl.BlockSpec((ROW_BLK, D), lambda i: (i, 0)),
        out_shape=jax.ShapeDtypeStruct((N_NODES, D), jnp.float32),
    )(parts, x, wrel_t, wroot_t, b2)


def kernel(x, edge_index, W_rel, W_root, b):
    src, dst = _edge_split(edge_index)
    parts = _sc_aggregate(x, src, dst)
    return _tc_combine(parts, x, W_rel, W_root, b.reshape(1, D))
